# baseline (speedup 1.0000x reference)
import functools

import jax
import jax.numpy as jnp
from jax import lax
from jax.experimental import pallas as pl
from jax.experimental.pallas import tpu as pltpu
from jax.experimental.pallas import tpu_sc as plsc

D = 64
B = 4096
L = 200

NC = 2
NS = 16
NW = NC * NS
ROWS_W = B // NW
G0 = 128
G1 = L - G0
NCHUNK = ROWS_W
NBUF = 4
NOUTER = NCHUNK // NBUF
SCALE = 8.0


@functools.partial(
    pl.kernel,
    out_type=jax.ShapeDtypeStruct((B, L, D), jnp.float32),
    mesh=plsc.VectorSubcoreMesh(core_axis_name="c", subcore_axis_name="s"),
    scratch_types=[
        pltpu.VMEM((ROWS_W, L), jnp.int32),
        [pltpu.VMEM((L, D), jnp.float32) for _ in range(NBUF)],
        [pltpu.SemaphoreType.DMA for _ in range(NBUF)],
        [pltpu.SemaphoreType.DMA for _ in range(NBUF)],
    ],
    compiler_params=pltpu.CompilerParams(use_tc_tiling_on_sc=False),
)
def _embed(x_hbm, table_hbm, out_hbm, idx_v, bufs, gsems, osems):
    wid = lax.axis_index("s") * NC + lax.axis_index("c")
    row_base = wid * ROWS_W
    pltpu.sync_copy(x_hbm.at[pl.ds(row_base, ROWS_W)], idx_v)

    def gather_descs(c, buf, sem):
        return [
            pltpu.make_async_copy(
                table_hbm.at[idx_v.at[c, pl.ds(0, G0)]],
                buf.at[pl.ds(0, G0)],
                sem,
            ),
            pltpu.make_async_copy(
                table_hbm.at[idx_v.at[c, pl.ds(G0, G1)]],
                buf.at[pl.ds(G0, G1)],
                sem,
            ),
        ]

    def start_gather(c, buf, sem):
        for cp in gather_descs(c, buf, sem):
            cp.start()

    def wait_gather(c, buf, sem):
        for cp in gather_descs(c, buf, sem):
            cp.wait()

    def start_out(c, buf, sem):
        pltpu.async_copy(buf, out_hbm.at[row_base + c], sem)

    def wait_out(buf, sem):
        pltpu.make_async_copy(buf, out_hbm.at[0], sem).wait()

    def scale(buf):
        def scale_row(i, carry):
            for j in range(D // 16):
                buf[i, pl.ds(j * 16, 16)] = buf[i, pl.ds(j * 16, 16)] * SCALE
            return carry

        lax.fori_loop(0, L, scale_row, 0)

    for b in range(NBUF - 1):
        start_gather(b, bufs[b], gsems[b])

    def outer(p, carry):
        for b in range(NBUF):
            g = p * NBUF + b
            nb = (b + NBUF - 1) % NBUF
            nxt = g + NBUF - 1

            @pl.when(nxt < NCHUNK)
            def _issue():
                if b == 0:

                    @pl.when(p > 0)
                    def _():
                        wait_out(bufs[nb], osems[nb])

                else:
                    wait_out(bufs[nb], osems[nb])
                start_gather(nxt, bufs[nb], gsems[nb])

            wait_gather(g, bufs[b], gsems[b])
            scale(bufs[b])
            start_out(g, bufs[b], osems[b])
        return carry

    lax.fori_loop(0, NOUTER, outer, 0)

    for b in range(NBUF):
        wait_out(bufs[b], osems[b])


def kernel(x, table):
    return _embed(x.astype(jnp.int32), table)
